# baseline (speedup 1.0000x reference)
import functools

import jax
import jax.numpy as jnp
from jax import lax
from jax.experimental import pallas as pl
from jax.experimental.pallas import tpu as pltpu
from jax.experimental.pallas import tpu_sc as plsc

_VOCAB = 100000
_EMBED = 64
_BATCH = 4096
_FIELDS = 26

_NC = 2
_NS = 16
_NW = _NC * _NS

_BPW = _BATCH // _NW
_NB = 7
_FPR = (7, 7, 7, 5)
_OFF = (0, 7, 14, 21)


@jax.jit
def _sc_gather(feat_t, table):
  mesh = plsc.VectorSubcoreMesh(core_axis_name="c", subcore_axis_name="s")

  @functools.partial(
      pl.kernel,
      mesh=mesh,
      compiler_params=pltpu.CompilerParams(use_tc_tiling_on_sc=False),
      out_type=jax.ShapeDtypeStruct((_BATCH, _FIELDS, _EMBED), jnp.float32),
      scratch_types=[
          pltpu.VMEM((_FIELDS, _BPW), jnp.int32),
          pltpu.VMEM((2, _NB, _BPW, _EMBED), jnp.float32),
          pltpu.SemaphoreType.DMA,
          pltpu.SemaphoreType.DMA,
          pltpu.SemaphoreType.DMA,
          pltpu.SemaphoreType.DMA,
      ],
  )
  def k(table_hbm, idx_hbm, out_hbm, idx_v, stage, g0, g1, o0, o1):
    sem_g = (g0, g1)
    sem_o = (o0, o1)
    wid = lax.axis_index("s") * _NC + lax.axis_index("c")
    base = wid * _BPW
    pltpu.sync_copy(idx_hbm.at[:, pl.ds(base, _BPW)], idx_v)

    def fire_gathers(r, p):
      for b in range(_FPR[r]):
        pltpu.async_copy(
            table_hbm.at[idx_v.at[_OFF[r] + b]],
            stage.at[p, b],
            sem_g[p],
        )

    def drain_gathers(r, p):
      for b in range(_FPR[r]):
        pltpu.make_async_copy(
            table_hbm.at[pl.ds(0, _BPW)], stage.at[p, b], sem_g[p]
        ).wait()

    def fire_outs(r, p):
      for b in range(_FPR[r]):
        pltpu.async_copy(
            stage.at[p, b],
            out_hbm.at[pl.ds(base, _BPW), _OFF[r] + b],
            sem_o[p],
        )

    def drain_outs(r, p):
      for b in range(_FPR[r]):
        pltpu.make_async_copy(
            stage.at[p, b], out_hbm.at[pl.ds(base, _BPW), 0], sem_o[p]
        ).wait()

    fire_gathers(0, 0)
    fire_gathers(1, 1)
    for r in range(len(_FPR)):
      p = r % 2
      drain_gathers(r, p)
      fire_outs(r, p)
      if r + 2 < len(_FPR):
        drain_outs(r, p)
        fire_gathers(r + 2, p)
    drain_outs(2, 0)
    drain_outs(3, 1)

  return k(table, feat_t)


def kernel(feat, emb_feat):
  return _sc_gather(feat.T, emb_feat)
